# baseline (speedup 1.0000x reference)
import functools

import jax
import jax.numpy as jnp
from jax import lax
from jax.experimental import pallas as pl
from jax.experimental.pallas import tpu as pltpu
from jax.experimental.pallas import tpu_sc as plsc

_R = 4


@functools.lru_cache(maxsize=None)
def _build(B, S, V, D):
    info = plsc.get_sparse_core_info()
    NC, NS, L = info.num_cores, info.num_subcores, info.num_lanes
    NW = NC * NS
    BLK = 128
    n_blocks = B // BLK
    blocks_per_w = n_blocks // NW
    assert n_blocks % NW == 0 and D == 32 and L == 16
    n_units = blocks_per_w * S
    assert n_units % _R == 0
    groups = BLK // L
    idx_rows_w = blocks_per_w * BLK * S // 128

    mesh = plsc.VectorSubcoreMesh(core_axis_name="c", subcore_axis_name="s")

    @functools.partial(
        pl.kernel,
        mesh=mesh,
        out_type=jax.ShapeDtypeStruct((S, D, B), jnp.float32),
        scratch_types=[
            pltpu.VMEM((idx_rows_w, 128), jnp.int32),
            pltpu.VMEM((_R, BLK), jnp.int32),
            pltpu.VMEM((_R, BLK), jnp.int32),
            pltpu.VMEM((_R, BLK, 128), jnp.float32),
            pltpu.VMEM((_R, D, BLK), jnp.float32),
            pltpu.SemaphoreType.DMA((_R,)),
            pltpu.SemaphoreType.DMA((_R,)),
        ],
        compiler_params=pltpu.CompilerParams(
            use_tc_tiling_on_sc=True, needs_layout_passes=False),
    )
    def gather_kernel(idx_hbm, tab_hbm, out_hbm, idxv, iv2, qv, v, w, gsem, wsem):
        wid = lax.axis_index("s") * NC + lax.axis_index("c")
        blk0 = wid * blocks_per_w
        lanes50 = [(lax.iota(jnp.int32, L) + kb * L) * S for kb in range(groups)]

        pltpu.sync_copy(idx_hbm.at[pl.ds(wid * idx_rows_w, idx_rows_w), :], idxv)

        def prep(base, slot):
            for kb in range(groups):
                pos = lanes50[kb] + base
                orig = plsc.load_gather(
                    idxv, [lax.shift_right_logical(pos, 7),
                           lax.bitwise_and(pos, 127)])
                iv2[slot, pl.ds(kb * L, L)] = lax.shift_right_logical(orig, 2)
                qv[slot, pl.ds(kb * L, L)] = lax.shift_left(
                    lax.bitwise_and(orig, 3), 5)

        def start_gather(slot):
            pltpu.async_copy(tab_hbm.at[iv2.at[slot]], v.at[slot], gsem.at[slot])

        def wait_gather(slot):
            pltpu.make_async_copy(
                tab_hbm.at[iv2.at[slot]], v.at[slot], gsem.at[slot]).wait()

        def transpose(slot):
            for kb in range(groups):
                qcol = qv[slot, pl.ds(kb * L, L)]
                rows = lax.iota(jnp.int32, L) + kb * L
                slots = jnp.full((L,), slot, jnp.int32)
                for dh in range(0, D, 32):
                    vals = [plsc.load_gather(v, [slots, rows, qcol + d])
                            for d in range(dh, dh + 32)]
                    for i, d in enumerate(range(dh, dh + 32)):
                        w[slot, d, pl.ds(kb * L, L)] = vals[i]

        def write_copy(slot, s, blk):
            return pltpu.make_async_copy(
                w.at[slot],
                out_hbm.at[s, :, pl.ds((blk0 + blk) * BLK, BLK)],
                wsem.at[slot],
            )

        for u in range(_R):
            prep(u, u)
            start_gather(u)

        def body(u2, carry):
            for j in range(_R):
                u = _R * u2 + j
                s = lax.rem(u, S)
                blk = lax.div(u, S)

                wait_gather(j)

                @pl.when(u2 >= 1)
                def _():
                    write_copy(j, s, blk).wait()

                transpose(j)

                u4 = u + _R

                @pl.when(u4 < n_units)
                def _():
                    s4 = lax.rem(u4, S)
                    blk4 = lax.div(u4, S)
                    prep(blk4 * (BLK * S) + s4, j)
                    start_gather(j)

                write_copy(slot=j, s=s, blk=blk).start()
            return carry

        lax.fori_loop(0, n_units // _R, body, 0)
        for j in range(_R):
            write_copy(j, 0, 0).wait()

    return gather_kernel


def kernel(token_ids, embedding_matrix):
    S0, S1 = token_ids.shape
    V, D = embedding_matrix.shape
    B = S0 * S1
    idx = token_ids
    if idx.dtype != jnp.int32:
        idx = idx.astype(jnp.int32)
    idx2d = idx.reshape(B // 128, 128)
    t2 = embedding_matrix.reshape(V // 4, D * 4)
    out = _build(S0, S1, V, D)(idx2d, t2)
    return jnp.transpose(out, (2, 0, 1))


# speedup vs baseline: 1.0119x; 1.0119x over previous
import functools

import jax
import jax.numpy as jnp
from jax import lax
from jax.experimental import pallas as pl
from jax.experimental.pallas import tpu as pltpu
from jax.experimental.pallas import tpu_sc as plsc

_R = 4


@functools.lru_cache(maxsize=None)
def _build(B, S, V, D):
    info = plsc.get_sparse_core_info()
    NC, NS, L = info.num_cores, info.num_subcores, info.num_lanes
    NW = NC * NS
    BLK = 128
    n_blocks = B // BLK
    blocks_per_w = n_blocks // NW
    assert n_blocks % NW == 0 and D == 32 and L == 16
    n_units = blocks_per_w * S
    assert n_units % _R == 0
    groups = BLK // L
    idx_rows_w = blocks_per_w * BLK * S // 128

    mesh = plsc.VectorSubcoreMesh(core_axis_name="c", subcore_axis_name="s")

    @functools.partial(
        pl.kernel,
        mesh=mesh,
        out_type=jax.ShapeDtypeStruct((S, D, B), jnp.float32),
        scratch_types=[
            pltpu.VMEM((idx_rows_w, 128), jnp.int32),
            pltpu.VMEM((_R, BLK), jnp.int32),
            pltpu.VMEM((_R, BLK), jnp.int32),
            pltpu.VMEM((_R, BLK, 128), jnp.float32),
            pltpu.VMEM((_R, D, BLK), jnp.float32),
            pltpu.SemaphoreType.DMA((_R,)),
            pltpu.SemaphoreType.DMA((_R,)),
        ],
        compiler_params=pltpu.CompilerParams(
            use_tc_tiling_on_sc=True, needs_layout_passes=False),
    )
    def gather_kernel(idx_hbm, tab_hbm, out_hbm, idxv, iv2, qv, v, w, gsem, wsem):
        wid = lax.axis_index("s") * NC + lax.axis_index("c")
        blk0 = wid * blocks_per_w
        lanes50 = [(lax.iota(jnp.int32, L) + kb * L) * S for kb in range(groups)]

        pltpu.sync_copy(idx_hbm.at[pl.ds(wid * idx_rows_w, idx_rows_w), :], idxv)

        def prep(base, slot):
            for kb in range(groups):
                pos = lanes50[kb] + base
                orig = plsc.load_gather(
                    idxv, [lax.shift_right_logical(pos, 7),
                           lax.bitwise_and(pos, 127)])
                iv2[slot, pl.ds(kb * L, L)] = lax.shift_right_logical(orig, 2)
                qv[slot, pl.ds(kb * L, L)] = lax.shift_left(
                    lax.bitwise_and(orig, 3), 5)

        def start_gather(slot):
            pltpu.async_copy(tab_hbm.at[iv2.at[slot]], v.at[slot], gsem.at[slot])

        def wait_gather(slot):
            pltpu.make_async_copy(
                tab_hbm.at[iv2.at[slot]], v.at[slot], gsem.at[slot]).wait()

        def transpose(slot):
            for kb in range(groups):
                qcol = qv[slot, pl.ds(kb * L, L)]
                rows = lax.iota(jnp.int32, L) + kb * L
                slots = jnp.full((L,), slot, jnp.int32)
                for dh in range(0, D, 16):
                    vals = [plsc.load_gather(v, [slots, rows, qcol + d])
                            for d in range(dh, dh + 16)]
                    for i, d in enumerate(range(dh, dh + 16)):
                        w[slot, d, pl.ds(kb * L, L)] = vals[i]

        def write_copy(slot, s, blk):
            return pltpu.make_async_copy(
                w.at[slot],
                out_hbm.at[s, :, pl.ds((blk0 + blk) * BLK, BLK)],
                wsem.at[slot],
            )

        for u in range(_R):
            prep(u, u)
            start_gather(u)

        def body(u2, carry):
            for j in range(_R):
                u = _R * u2 + j
                s = lax.rem(u, S)
                blk = lax.div(u, S)

                wait_gather(j)

                @pl.when(u2 >= 1)
                def _():
                    write_copy(j, s, blk).wait()

                transpose(j)

                u4 = u + _R

                @pl.when(u4 < n_units)
                def _():
                    s4 = lax.rem(u4, S)
                    blk4 = lax.div(u4, S)
                    prep(blk4 * (BLK * S) + s4, j)
                    start_gather(j)

                write_copy(slot=j, s=s, blk=blk).start()
            return carry

        lax.fori_loop(0, n_units // _R, body, 0)
        for j in range(_R):
            write_copy(j, 0, 0).wait()

    return gather_kernel


def kernel(token_ids, embedding_matrix):
    S0, S1 = token_ids.shape
    V, D = embedding_matrix.shape
    B = S0 * S1
    idx = token_ids
    if idx.dtype != jnp.int32:
        idx = idx.astype(jnp.int32)
    idx2d = idx.reshape(B // 128, 128)
    t2 = embedding_matrix.reshape(V // 4, D * 4)
    out = _build(S0, S1, V, D)(idx2d, t2)
    return jnp.transpose(out, (2, 0, 1))
